# Initial kernel scaffold; baseline (speedup 1.0000x reference)
import jax
import jax.numpy as jnp
from jax.experimental import pallas as pl


def kernel(x, edge_index, W1l, b1, W1r, W2l, b2, W2r):
    raise NotImplementedError("write your pallas kernel here")


# speedup vs baseline: 3.1862x; 3.1862x over previous
import jax
import jax.numpy as jnp
from jax import lax
from jax.experimental import pallas as pl
from jax.experimental.pallas import tpu as pltpu, tpu_sc as plsc

_C = 128
_G = 8


def _round_up(a, b):
    return (a + b - 1) // b * b


def _make_sc_agg(NP, D, K, NC, NS):
    mesh = plsc.VectorSubcoreMesh(core_axis_name="c", subcore_axis_name="s")
    rpt = NP // NS

    def body(table, srcw, dstw, zbig, agg_out,
             src_v, dst_v, rows_v, accum, sem):
        c = lax.axis_index("c")
        s = lax.axis_index("s")
        wid = s * NC + c

        pltpu.sync_copy(zbig, accum.at[pl.ds(s * rpt, rpt)])
        plsc.subcore_barrier()

        def group(g, carry):
            pltpu.sync_copy(srcw.at[wid, pl.ds(g * _G, _G)], src_v)
            pltpu.sync_copy(dstw.at[wid, pl.ds(g * _G, _G)], dst_v)
            for j in range(_G):
                pltpu.async_copy(table.at[src_v.at[j]], rows_v, sem).wait()
                pltpu.sync_copy(rows_v, accum.at[dst_v.at[j]], add=True)
            return carry

        lax.fori_loop(0, K // _G, group, 0)
        plsc.subcore_barrier()

        sl = pl.ds(s * rpt, rpt)
        pltpu.sync_copy(accum.at[sl], agg_out.at[c, sl])

    return pl.kernel(
        body,
        out_type=jax.ShapeDtypeStruct((NC, NP, D), jnp.float32),
        mesh=mesh,
        scratch_types=[
            pltpu.VMEM((_G, _C), jnp.int32),
            pltpu.VMEM((_G, _C), jnp.int32),
            pltpu.VMEM((_C, D), jnp.float32),
            pltpu.VMEM_SHARED((NP, D), jnp.float32),
            pltpu.SemaphoreType.DMA,
        ],
    )


def _make_sc_deg(NP, D, K, NC, NS):
    mesh = plsc.VectorSubcoreMesh(core_axis_name="c", subcore_axis_name="s")
    rpt = NP // NS

    def body(dstw, zbig, ones_hbm, deg_out, dst_v, ones_v, accum):
        c = lax.axis_index("c")
        s = lax.axis_index("s")
        wid = s * NC + c

        pltpu.sync_copy(zbig, accum.at[pl.ds(s * rpt, rpt)])
        pltpu.sync_copy(ones_hbm, ones_v)
        plsc.subcore_barrier()

        def group(g, carry):
            pltpu.sync_copy(dstw.at[wid, pl.ds(g * _G, _G)], dst_v)
            for j in range(_G):
                pltpu.sync_copy(ones_v, accum.at[dst_v.at[j]], add=True)
            return carry

        lax.fori_loop(0, K // _G, group, 0)
        plsc.subcore_barrier()

        sl = pl.ds(s * rpt, rpt)
        pltpu.sync_copy(accum.at[sl], deg_out.at[c, sl])

    return pl.kernel(
        body,
        out_type=jax.ShapeDtypeStruct((NC, NP, D), jnp.float32),
        mesh=mesh,
        scratch_types=[
            pltpu.VMEM((_G, _C), jnp.int32),
            pltpu.VMEM((_C, D), jnp.float32),
            pltpu.VMEM_SHARED((NP, D), jnp.float32),
        ],
    )


def _make_tc_layer(NP, D, relu):
    BR = 256

    def body(a0, a1, d0, d1, x, wlT, wrT, b, out):
        agg = a0[...] + a1[...]
        degc = jnp.maximum(d0[:, 0:1] + d1[:, 0:1], 1.0)
        m = agg / degc
        h = (jnp.dot(m, wlT[...], preferred_element_type=jnp.float32)
             + jnp.dot(x[...], wrT[...], preferred_element_type=jnp.float32)
             + b[...])
        out[...] = jnp.maximum(h, 0.0) if relu else h

    row = lambda i: (i, 0)
    fixed = lambda i: (0, 0)
    return pl.pallas_call(
        body,
        grid=(NP // BR,),
        in_specs=[
            pl.BlockSpec((BR, D), row),
            pl.BlockSpec((BR, D), row),
            pl.BlockSpec((BR, D), row),
            pl.BlockSpec((BR, D), row),
            pl.BlockSpec((BR, D), row),
            pl.BlockSpec((D, D), fixed),
            pl.BlockSpec((D, D), fixed),
            pl.BlockSpec((1, D), fixed),
        ],
        out_specs=pl.BlockSpec((BR, D), row),
        out_shape=jax.ShapeDtypeStruct((NP, D), jnp.float32),
    )


def kernel(x, edge_index, W1l, b1, W1r, W2l, b2, W2r):
    N, D = x.shape
    E = edge_index.shape[1]
    info = plsc.get_sparse_core_info()
    NC, NS = info.num_cores, info.num_subcores
    NW = NC * NS

    rpt = _round_up(-(-(N + 1) // NS), 64)
    NP = rpt * NS
    K = _round_up(-(-E // (NW * _C)), _G)
    EP = NW * K * _C

    src = edge_index[0]
    dst = edge_index[1]
    padfill = jnp.full((EP - E,), N, jnp.int32)
    srcp = jnp.concatenate([src, padfill]).reshape(NW, K, _C)
    dstp = jnp.concatenate([dst, padfill]).reshape(NW, K, _C)
    xp = jnp.pad(x, ((0, NP - N), (0, 0)))
    zbig = jnp.zeros((rpt, D), jnp.float32)
    ones_rows = jnp.zeros((_C, D), jnp.float32).at[:, 0].set(1.0)

    sc_agg = _make_sc_agg(NP, D, K, NC, NS)
    sc_deg = _make_sc_deg(NP, D, K, NC, NS)
    tc1 = _make_tc_layer(NP, D, relu=True)
    tc2 = _make_tc_layer(NP, D, relu=False)

    degP = sc_deg(dstp, zbig, ones_rows)
    aggP = sc_agg(xp, srcp, dstp, zbig)
    h = tc1(aggP[0], aggP[1], degP[0], degP[1], xp,
            W1l.T, W1r.T, b1.reshape(1, D))
    agg2P = sc_agg(h, srcp, dstp, zbig)
    out = tc2(agg2P[0], agg2P[1], degP[0], degP[1], h,
              W2l.T, W2r.T, b2.reshape(1, D))
    return out[:N]


# speedup vs baseline: 3.4816x; 1.0927x over previous
import jax
import jax.numpy as jnp
from jax import lax
from jax.experimental import pallas as pl
from jax.experimental.pallas import tpu as pltpu, tpu_sc as plsc

_C = 128
_G = 8


def _round_up(a, b):
    return (a + b - 1) // b * b


def _make_sc_agg(NP, D, K, NC, NS):
    mesh = plsc.VectorSubcoreMesh(core_axis_name="c", subcore_axis_name="s")
    rpt = NP // NS

    def body(table, srcw, dstw, zbig, agg_out,
             src_v, dst_v, rows_a, rows_b, accum, sem_a, sem_b):
        c = lax.axis_index("c")
        s = lax.axis_index("s")
        wid = s * NC + c

        pltpu.sync_copy(zbig, accum.at[pl.ds(s * rpt, rpt)])
        plsc.subcore_barrier()

        bufs = (rows_a, rows_b)
        sems = (sem_a, sem_b)

        def group(g, carry):
            pltpu.sync_copy(srcw.at[wid, pl.ds(g * _G, _G)], src_v)
            pltpu.sync_copy(dstw.at[wid, pl.ds(g * _G, _G)], dst_v)
            descs = [None, None]
            descs[0] = pltpu.async_copy(
                table.at[src_v.at[0]], bufs[0], sems[0])
            for j in range(_G):
                if j + 1 < _G:
                    b = (j + 1) % 2
                    descs[b] = pltpu.async_copy(
                        table.at[src_v.at[j + 1]], bufs[b], sems[b])
                descs[j % 2].wait()
                pltpu.sync_copy(bufs[j % 2], accum.at[dst_v.at[j]], add=True)
            return carry

        lax.fori_loop(0, K // _G, group, 0)
        plsc.subcore_barrier()

        sl = pl.ds(s * rpt, rpt)
        pltpu.sync_copy(accum.at[sl], agg_out.at[c, sl])

    return pl.kernel(
        body,
        out_type=jax.ShapeDtypeStruct((NC, NP, D), jnp.float32),
        mesh=mesh,
        scratch_types=[
            pltpu.VMEM((_G, _C), jnp.int32),
            pltpu.VMEM((_G, _C), jnp.int32),
            pltpu.VMEM((_C, D), jnp.float32),
            pltpu.VMEM((_C, D), jnp.float32),
            pltpu.VMEM_SHARED((NP, D), jnp.float32),
            pltpu.SemaphoreType.DMA,
            pltpu.SemaphoreType.DMA,
        ],
    )


def _make_sc_deg(NP, D, K, NC, NS):
    mesh = plsc.VectorSubcoreMesh(core_axis_name="c", subcore_axis_name="s")
    rpt = NP // NS

    def body(dstw, zbig, ones_hbm, deg_out, dst_v, ones_v, accum):
        c = lax.axis_index("c")
        s = lax.axis_index("s")
        wid = s * NC + c

        pltpu.sync_copy(zbig, accum.at[pl.ds(s * rpt, rpt)])
        pltpu.sync_copy(ones_hbm, ones_v)
        plsc.subcore_barrier()

        def group(g, carry):
            pltpu.sync_copy(dstw.at[wid, pl.ds(g * _G, _G)], dst_v)
            for j in range(_G):
                pltpu.sync_copy(ones_v, accum.at[dst_v.at[j]], add=True)
            return carry

        lax.fori_loop(0, K // _G, group, 0)
        plsc.subcore_barrier()

        sl = pl.ds(s * rpt, rpt)
        pltpu.sync_copy(accum.at[sl], deg_out.at[c, sl])

    return pl.kernel(
        body,
        out_type=jax.ShapeDtypeStruct((NC, NP, D), jnp.float32),
        mesh=mesh,
        scratch_types=[
            pltpu.VMEM((_G, _C), jnp.int32),
            pltpu.VMEM((_C, D), jnp.float32),
            pltpu.VMEM_SHARED((NP, D), jnp.float32),
        ],
    )


def _make_tc_layer(NP, D, relu):
    BR = 256

    def body(a0, a1, d0, d1, x, wlT, wrT, b, out):
        agg = a0[...] + a1[...]
        degc = jnp.maximum(d0[:, 0:1] + d1[:, 0:1], 1.0)
        m = agg / degc
        h = (jnp.dot(m, wlT[...], preferred_element_type=jnp.float32)
             + jnp.dot(x[...], wrT[...], preferred_element_type=jnp.float32)
             + b[...])
        out[...] = jnp.maximum(h, 0.0) if relu else h

    row = lambda i: (i, 0)
    fixed = lambda i: (0, 0)
    return pl.pallas_call(
        body,
        grid=(NP // BR,),
        in_specs=[
            pl.BlockSpec((BR, D), row),
            pl.BlockSpec((BR, D), row),
            pl.BlockSpec((BR, D), row),
            pl.BlockSpec((BR, D), row),
            pl.BlockSpec((BR, D), row),
            pl.BlockSpec((D, D), fixed),
            pl.BlockSpec((D, D), fixed),
            pl.BlockSpec((1, D), fixed),
        ],
        out_specs=pl.BlockSpec((BR, D), row),
        out_shape=jax.ShapeDtypeStruct((NP, D), jnp.float32),
    )


def kernel(x, edge_index, W1l, b1, W1r, W2l, b2, W2r):
    N, D = x.shape
    E = edge_index.shape[1]
    info = plsc.get_sparse_core_info()
    NC, NS = info.num_cores, info.num_subcores
    NW = NC * NS

    rpt = _round_up(-(-(N + 1) // NS), 64)
    NP = rpt * NS
    K = _round_up(-(-E // (NW * _C)), _G)
    EP = NW * K * _C

    src = edge_index[0]
    dst = edge_index[1]
    padfill = jnp.full((EP - E,), N, jnp.int32)
    srcp = jnp.concatenate([src, padfill]).reshape(NW, K, _C)
    dstp = jnp.concatenate([dst, padfill]).reshape(NW, K, _C)
    xp = jnp.pad(x, ((0, NP - N), (0, 0)))
    zbig = jnp.zeros((rpt, D), jnp.float32)
    ones_rows = jnp.zeros((_C, D), jnp.float32).at[:, 0].set(1.0)

    sc_agg = _make_sc_agg(NP, D, K, NC, NS)
    sc_deg = _make_sc_deg(NP, D, K, NC, NS)
    tc1 = _make_tc_layer(NP, D, relu=True)
    tc2 = _make_tc_layer(NP, D, relu=False)

    degP = sc_deg(dstp, zbig, ones_rows)
    aggP = sc_agg(xp, srcp, dstp, zbig)
    h = tc1(aggP[0], aggP[1], degP[0], degP[1], xp,
            W1l.T, W1r.T, b1.reshape(1, D))
    agg2P = sc_agg(h, srcp, dstp, zbig)
    out = tc2(agg2P[0], agg2P[1], degP[0], degP[1], h,
              W2l.T, W2r.T, b2.reshape(1, D))
    return out[:N]


# speedup vs baseline: 3.7621x; 1.0806x over previous
import jax
import jax.numpy as jnp
from jax import lax
from jax.experimental import pallas as pl
from jax.experimental.pallas import tpu as pltpu, tpu_sc as plsc

_C = 128
_G = 8


def _round_up(a, b):
    return (a + b - 1) // b * b


def _zero_fill(buf, D):
    def zrow(r, carry):
        for k in range(D // 16):
            buf[r, pl.ds(k * 16, 16)] = jnp.zeros((16,), jnp.float32)
        return carry

    lax.fori_loop(0, _C, zrow, 0)


def _make_sc_agg(NP, D, NG0, NG1, NC, NS):
    mesh = plsc.VectorSubcoreMesh(core_axis_name="c", subcore_axis_name="s")
    rpt = NP // NS

    def body(table, srcw0, srcw1, dstw0, dstw1, agg_out,
             src_v, dst_v, buf_a, buf_b, accum, sem_a, sem_b):
        c = lax.axis_index("c")
        s = lax.axis_index("s")

        _zero_fill(buf_a, D)
        for t in range(rpt // _C):
            pltpu.sync_copy(buf_a, accum.at[pl.ds(s * rpt + t * _C, _C)])
        plsc.subcore_barrier()

        bufs = (buf_a, buf_b)
        sems = (sem_a, sem_b)

        def make_group(srcw, dstw):
            def group(g, carry):
                pltpu.sync_copy(srcw.at[s, pl.ds(g * _G, _G)], src_v)
                pltpu.sync_copy(dstw.at[s, pl.ds(g * _G, _G)], dst_v)
                descs = [None, None]
                descs[0] = pltpu.async_copy(
                    table.at[src_v.at[0]], bufs[0], sems[0])
                for j in range(_G):
                    if j + 1 < _G:
                        b = (j + 1) % 2
                        descs[b] = pltpu.async_copy(
                            table.at[src_v.at[j + 1]], bufs[b], sems[b])
                    descs[j % 2].wait()
                    pltpu.sync_copy(
                        bufs[j % 2], accum.at[dst_v.at[j]], add=True)
                return carry

            return group

        @pl.when(c == 0)
        def _():
            lax.fori_loop(0, NG0, make_group(srcw0, dstw0), 0)

        @pl.when(c == 1)
        def _():
            lax.fori_loop(0, NG1, make_group(srcw1, dstw1), 0)

        plsc.subcore_barrier()

        sl = pl.ds(s * rpt, rpt)
        pltpu.sync_copy(accum.at[sl], agg_out.at[c, sl])

    return pl.kernel(
        body,
        out_type=jax.ShapeDtypeStruct((NC, NP, D), jnp.float32),
        mesh=mesh,
        scratch_types=[
            pltpu.VMEM((_G, _C), jnp.int32),
            pltpu.VMEM((_G, _C), jnp.int32),
            pltpu.VMEM((_C, D), jnp.float32),
            pltpu.VMEM((_C, D), jnp.float32),
            pltpu.VMEM_SHARED((NP, D), jnp.float32),
            pltpu.SemaphoreType.DMA,
            pltpu.SemaphoreType.DMA,
        ],
    )


def _make_sc_deg(NP, D, NGu, NC, NS):
    mesh = plsc.VectorSubcoreMesh(core_axis_name="c", subcore_axis_name="s")
    rpt = NP // NS

    def body(dstw, deg_out, dst_v, ones_v, accum):
        c = lax.axis_index("c")
        s = lax.axis_index("s")
        wid = s * NC + c

        _zero_fill(ones_v, D)
        for t in range(rpt // _C):
            pltpu.sync_copy(ones_v, accum.at[pl.ds(s * rpt + t * _C, _C)])
        onerow = jnp.where(lax.iota(jnp.int32, 16) == 0, 1.0, 0.0)

        def orow(r, carry):
            ones_v[r, pl.ds(0, 16)] = onerow
            return carry

        lax.fori_loop(0, _C, orow, 0)
        plsc.subcore_barrier()

        def group(g, carry):
            pltpu.sync_copy(dstw.at[wid, pl.ds(g * _G, _G)], dst_v)
            for j in range(_G):
                pltpu.sync_copy(ones_v, accum.at[dst_v.at[j]], add=True)
            return carry

        lax.fori_loop(0, NGu, group, 0)
        plsc.subcore_barrier()

        sl = pl.ds(s * rpt, rpt)
        pltpu.sync_copy(accum.at[sl], deg_out.at[c, sl])

    return pl.kernel(
        body,
        out_type=jax.ShapeDtypeStruct((NC, NP, D), jnp.float32),
        mesh=mesh,
        scratch_types=[
            pltpu.VMEM((_G, _C), jnp.int32),
            pltpu.VMEM((_C, D), jnp.float32),
            pltpu.VMEM_SHARED((NP, D), jnp.float32),
        ],
    )


def _make_tc_layer(NP, D, relu):
    BR = 256

    def body(a0, a1, d0, d1, x, wlT, wrT, b, out):
        agg = a0[...] + a1[...]
        degc = jnp.maximum(d0[:, 0:1] + d1[:, 0:1], 1.0)
        m = agg / degc
        h = (jnp.dot(m, wlT[...], preferred_element_type=jnp.float32)
             + jnp.dot(x[...], wrT[...], preferred_element_type=jnp.float32)
             + b[...])
        out[...] = jnp.maximum(h, 0.0) if relu else h

    row = lambda i: (i, 0)
    fixed = lambda i: (0, 0)
    return pl.pallas_call(
        body,
        grid=(NP // BR,),
        in_specs=[
            pl.BlockSpec((BR, D), row),
            pl.BlockSpec((BR, D), row),
            pl.BlockSpec((BR, D), row),
            pl.BlockSpec((BR, D), row),
            pl.BlockSpec((BR, D), row),
            pl.BlockSpec((D, D), fixed),
            pl.BlockSpec((D, D), fixed),
            pl.BlockSpec((1, D), fixed),
        ],
        out_specs=pl.BlockSpec((BR, D), row),
        out_shape=jax.ShapeDtypeStruct((NP, D), jnp.float32),
    )


def kernel(x, edge_index, W1l, b1, W1r, W2l, b2, W2r):
    N, D = x.shape
    E = edge_index.shape[1]
    info = plsc.get_sparse_core_info()
    NC, NS = info.num_cores, info.num_subcores
    NW = NC * NS

    rpt = _round_up(-(-(N + 1) // NS), _C)
    NP = rpt * NS

    NGT = _round_up(-(-E // (NS * _G * _C)), 2)
    NG1 = NGT // 4
    NG0 = NGT - NG1
    NGu = NGT // 2

    ew0 = NG0 * _G * _C
    EP = NS * (ew0 + NG1 * _G * _C)
    E0 = NS * ew0

    pad = jnp.full((EP - E,), N, jnp.int32)
    srcf = jnp.concatenate([edge_index[0], pad])
    dstf = jnp.concatenate([edge_index[1], pad])
    srcw0 = srcf[:E0].reshape(NS, NG0 * _G, _C)
    srcw1 = srcf[E0:].reshape(NS, NG1 * _G, _C)
    dstw0 = dstf[:E0].reshape(NS, NG0 * _G, _C)
    dstw1 = dstf[E0:].reshape(NS, NG1 * _G, _C)
    dstu = dstf.reshape(NW, NGu * _G, _C)
    xp = jnp.pad(x, ((0, NP - N), (0, 0)))

    sc_agg = _make_sc_agg(NP, D, NG0, NG1, NC, NS)
    sc_deg = _make_sc_deg(NP, D, NGu, NC, NS)
    tc1 = _make_tc_layer(NP, D, relu=True)
    tc2 = _make_tc_layer(NP, D, relu=False)

    degP = sc_deg(dstu)
    aggP = sc_agg(xp, srcw0, srcw1, dstw0, dstw1)
    h = tc1(aggP[0], aggP[1], degP[0], degP[1], xp,
            W1l.T, W1r.T, b1.reshape(1, D))
    agg2P = sc_agg(h, srcw0, srcw1, dstw0, dstw1)
    out = tc2(agg2P[0], agg2P[1], degP[0], degP[1], h,
              W2l.T, W2r.T, b2.reshape(1, D))
    return out[:N]
